# baseline (speedup 1.0000x reference)
import jax
import jax.numpy as jnp
from jax.experimental import pallas as pl
from jax.experimental.pallas import tpu as pltpu

B, IN, Q, M, OUT = 4096, 2048, 6, 512, 16
COEFS = (-3.0, -0.834, -0.248, 0.248, 0.834)

B_T = 256
IN_T = 256
FL_T = IN_T * Q


def _stats_body(x_ref, mean_ref, var_ref):
    xb = x_ref[...]
    s = jnp.sum(xb, axis=0)
    ss = jnp.sum(xb * xb, axis=0)
    mean = s * (1.0 / B)
    mean_ref[0, :] = mean
    var_ref[0, :] = ss * (1.0 / B) - mean * mean


def _stats(x):
    return pl.pallas_call(
        _stats_body,
        grid=(IN // 512,),
        in_specs=[pl.BlockSpec((B, 512), lambda i: (0, i))],
        out_specs=[
            pl.BlockSpec((1, 512), lambda i: (0, i)),
            pl.BlockSpec((1, 512), lambda i: (0, i)),
        ],
        out_shape=[
            jax.ShapeDtypeStruct((1, IN), jnp.float32),
            jax.ShapeDtypeStruct((1, IN), jnp.float32),
        ],
    )(x)


def _main_body(x_ref, scale_ref, off_ref, sf_ref, n1_ref, n2_ref, e_ref,
               wmx_ref, flat_ref, acc_ref, g1_ref, g2_ref):
    ii = pl.program_id(1)
    normed = x_ref[...] * scale_ref[0, :] + off_ref[0, :]
    nb = normed.astype(jnp.bfloat16)
    expanded = jnp.dot(nb, e_ref[...], preferred_element_type=jnp.float32)
    flat = jnp.maximum(expanded - sf_ref[0, :], 0.0)
    flat_ref[...] = flat
    pacc = jnp.dot(flat.astype(jnp.bfloat16), wmx_ref[...].astype(jnp.bfloat16),
                   preferred_element_type=jnp.float32)
    rows = jax.lax.broadcasted_iota(jnp.int32, (IN_T, M), 0) + ii * IN_T
    p1 = (rows == n1_ref[...]).astype(jnp.bfloat16)
    p2 = (rows == n2_ref[...]).astype(jnp.bfloat16)
    g1p = jnp.dot(nb, p1, preferred_element_type=jnp.float32)
    g2p = jnp.dot(nb, p2, preferred_element_type=jnp.float32)

    @pl.when(ii == 0)
    def _():
        acc_ref[...] = jnp.zeros_like(acc_ref)
        g1_ref[...] = jnp.zeros_like(g1_ref)
        g2_ref[...] = jnp.zeros_like(g2_ref)

    acc_ref[...] += pacc
    g1_ref[...] += g1p
    g2_ref[...] += g2p


def _main(x, scale, off, shift_flat, n1, n2, e_mat, w_mx):
    return pl.pallas_call(
        _main_body,
        grid=(B // B_T, IN // IN_T),
        in_specs=[
            pl.BlockSpec((B_T, IN_T), lambda ib, ii: (ib, ii)),
            pl.BlockSpec((1, IN_T), lambda ib, ii: (0, ii)),
            pl.BlockSpec((1, IN_T), lambda ib, ii: (0, ii)),
            pl.BlockSpec((1, FL_T), lambda ib, ii: (0, ii)),
            pl.BlockSpec((1, M), lambda ib, ii: (0, 0)),
            pl.BlockSpec((1, M), lambda ib, ii: (0, 0)),
            pl.BlockSpec((IN_T, FL_T), lambda ib, ii: (0, 0)),
            pl.BlockSpec((FL_T, OUT), lambda ib, ii: (ii, 0)),
        ],
        out_specs=[
            pl.BlockSpec((B_T, FL_T), lambda ib, ii: (ib, ii)),
            pl.BlockSpec((B_T, OUT), lambda ib, ii: (ib, 0)),
            pl.BlockSpec((B_T, M), lambda ib, ii: (ib, 0)),
            pl.BlockSpec((B_T, M), lambda ib, ii: (ib, 0)),
        ],
        out_shape=[
            jax.ShapeDtypeStruct((B, IN * Q), jnp.float32),
            jax.ShapeDtypeStruct((B, OUT), jnp.float32),
            jax.ShapeDtypeStruct((B, M), jnp.float32),
            jax.ShapeDtypeStruct((B, M), jnp.float32),
        ],
        compiler_params=pltpu.CompilerParams(
            dimension_semantics=("parallel", "arbitrary"),
        ),
    )(x, scale, off, shift_flat, n1, n2, e_mat, w_mx)


def _combine_body(g1_ref, g2_ref, s1_ref, s2_ref, acc_ref, wmin_ref, b_ref,
                  minx_ref, out_ref):
    d1 = jnp.maximum(g1_ref[...] - s1_ref[0, :], 0.0)
    d2 = jnp.maximum(g2_ref[...] - s2_ref[0, :], 0.0)
    mn = jnp.minimum(d1, d2)
    minx_ref[...] = mn
    out_ref[...] = (acc_ref[...]
                    + jnp.dot(mn, wmin_ref[...], preferred_element_type=jnp.float32)
                    + b_ref[0, 0])


def _combine(g1, g2, s1, s2, acc, w_min, biases):
    return pl.pallas_call(
        _combine_body,
        grid=(B // B_T,),
        in_specs=[
            pl.BlockSpec((B_T, M), lambda ib: (ib, 0)),
            pl.BlockSpec((B_T, M), lambda ib: (ib, 0)),
            pl.BlockSpec((1, M), lambda ib: (0, 0)),
            pl.BlockSpec((1, M), lambda ib: (0, 0)),
            pl.BlockSpec((B_T, OUT), lambda ib: (ib, 0)),
            pl.BlockSpec((M, OUT), lambda ib: (0, 0)),
            pl.BlockSpec((1, 1), lambda ib: (0, 0)),
        ],
        out_specs=[
            pl.BlockSpec((B_T, M), lambda ib: (ib, 0)),
            pl.BlockSpec((B_T, OUT), lambda ib: (ib, 0)),
        ],
        out_shape=[
            jax.ShapeDtypeStruct((B, M), jnp.float32),
            jax.ShapeDtypeStruct((B, OUT), jnp.float32),
        ],
    )(g1, g2, s1, s2, acc, w_min, biases)


def kernel(x, init_struct, beta, gamma, w, biases, chosen_index):
    mean2, var2 = _stats(x)
    mean, var = mean2[0], var2[0]
    inv = jax.lax.rsqrt(var + 0.001)
    scale = gamma * inv
    off = beta - mean * scale
    shift = jnp.concatenate(
        [jnp.zeros((1, IN), jnp.float32)]
        + [((c * var + mean) * gamma - beta)[None, :] for c in COEFS], axis=0)
    shift_flat = jnp.transpose(shift).reshape(1, IN * Q)
    n1 = chosen_index[:, 1]
    q1 = chosen_index[:, 2]
    n2 = chosen_index[:, 3]
    q2 = chosen_index[:, 4]
    s1 = shift[q1, n1][None, :]
    s2 = shift[q2, n2][None, :]
    e_mat = (jnp.arange(FL_T)[None, :] // Q == jnp.arange(IN_T)[:, None]
             ).astype(jnp.bfloat16)
    flat, acc, g1, g2 = _main(x, scale[None, :], off[None, :], shift_flat,
                              n1[None, :], n2[None, :], e_mat, w[:IN * Q])
    min_x, output = _combine(g1, g2, s1, s2, acc, w[IN * Q:], biases.reshape(1, 1))
    max_x = flat.reshape(B, IN, Q)
    return (output, w, max_x, min_x)


# speedup vs baseline: 2.7139x; 2.7139x over previous
import jax
import jax.numpy as jnp
from jax.experimental import pallas as pl
from jax.experimental.pallas import tpu as pltpu

B, IN, Q, M, OUT = 4096, 2048, 6, 512, 16
COEFS = (-3.0, -0.834, -0.248, 0.248, 0.834)

B_T = 256
IN_T = 256


def _stats_body(x_ref, mean_ref, var_ref):
    xb = x_ref[...]
    s = jnp.sum(xb, axis=0)
    ss = jnp.sum(xb * xb, axis=0)
    mean = s * (1.0 / B)
    mean_ref[0, :] = mean
    var_ref[0, :] = ss * (1.0 / B) - mean * mean


def _stats(x):
    return pl.pallas_call(
        _stats_body,
        grid=(IN // 512,),
        in_specs=[pl.BlockSpec((B, 512), lambda i: (0, i))],
        out_specs=[
            pl.BlockSpec((1, 512), lambda i: (0, i)),
            pl.BlockSpec((1, 512), lambda i: (0, i)),
        ],
        out_shape=[
            jax.ShapeDtypeStruct((1, IN), jnp.float32),
            jax.ShapeDtypeStruct((1, IN), jnp.float32),
        ],
    )(x)


def _main_body(x_ref, scale_ref, off_ref, shq_ref, n1_ref, n2_ref,
               wq_ref, mx_ref, acc_ref, g1_ref, g2_ref):
    ii = pl.program_id(1)
    normed = x_ref[...] * scale_ref[0, :] + off_ref[0, :]
    nb = normed.astype(jnp.bfloat16)
    pacc = jnp.zeros((B_T, OUT), jnp.float32)
    for q in range(Q):
        rq = jnp.maximum(normed - shq_ref[q, :], 0.0)
        mx_ref[q] = rq
        pacc += jnp.dot(rq.astype(jnp.bfloat16), wq_ref[q].astype(jnp.bfloat16),
                        preferred_element_type=jnp.float32)
    rows = jax.lax.broadcasted_iota(jnp.int32, (IN_T, M), 0) + ii * IN_T
    p1 = (rows == n1_ref[...]).astype(jnp.bfloat16)
    p2 = (rows == n2_ref[...]).astype(jnp.bfloat16)
    g1p = jnp.dot(nb, p1, preferred_element_type=jnp.float32)
    g2p = jnp.dot(nb, p2, preferred_element_type=jnp.float32)

    @pl.when(ii == 0)
    def _():
        acc_ref[...] = jnp.zeros_like(acc_ref)
        g1_ref[...] = jnp.zeros_like(g1_ref)
        g2_ref[...] = jnp.zeros_like(g2_ref)

    acc_ref[...] += pacc
    g1_ref[...] += g1p
    g2_ref[...] += g2p


def _main(x, scale, off, shiftq, n1, n2, w_q):
    return pl.pallas_call(
        _main_body,
        grid=(B // B_T, IN // IN_T),
        in_specs=[
            pl.BlockSpec((B_T, IN_T), lambda ib, ii: (ib, ii)),
            pl.BlockSpec((1, IN_T), lambda ib, ii: (0, ii)),
            pl.BlockSpec((1, IN_T), lambda ib, ii: (0, ii)),
            pl.BlockSpec((Q, IN_T), lambda ib, ii: (0, ii)),
            pl.BlockSpec((1, M), lambda ib, ii: (0, 0)),
            pl.BlockSpec((1, M), lambda ib, ii: (0, 0)),
            pl.BlockSpec((Q, IN_T, OUT), lambda ib, ii: (0, ii, 0)),
        ],
        out_specs=[
            pl.BlockSpec((Q, B_T, IN_T), lambda ib, ii: (0, ib, ii)),
            pl.BlockSpec((B_T, OUT), lambda ib, ii: (ib, 0)),
            pl.BlockSpec((B_T, M), lambda ib, ii: (ib, 0)),
            pl.BlockSpec((B_T, M), lambda ib, ii: (ib, 0)),
        ],
        out_shape=[
            jax.ShapeDtypeStruct((Q, B, IN), jnp.float32),
            jax.ShapeDtypeStruct((B, OUT), jnp.float32),
            jax.ShapeDtypeStruct((B, M), jnp.float32),
            jax.ShapeDtypeStruct((B, M), jnp.float32),
        ],
        compiler_params=pltpu.CompilerParams(
            dimension_semantics=("parallel", "arbitrary"),
        ),
    )(x, scale, off, shiftq, n1, n2, w_q)


def _combine_body(g1_ref, g2_ref, s1_ref, s2_ref, acc_ref, wmin_ref, b_ref,
                  minx_ref, out_ref):
    d1 = jnp.maximum(g1_ref[...] - s1_ref[0, :], 0.0)
    d2 = jnp.maximum(g2_ref[...] - s2_ref[0, :], 0.0)
    mn = jnp.minimum(d1, d2)
    minx_ref[...] = mn
    out_ref[...] = (acc_ref[...]
                    + jnp.dot(mn, wmin_ref[...], preferred_element_type=jnp.float32)
                    + b_ref[0, 0])


def _combine(g1, g2, s1, s2, acc, w_min, biases):
    return pl.pallas_call(
        _combine_body,
        grid=(B // B_T,),
        in_specs=[
            pl.BlockSpec((B_T, M), lambda ib: (ib, 0)),
            pl.BlockSpec((B_T, M), lambda ib: (ib, 0)),
            pl.BlockSpec((1, M), lambda ib: (0, 0)),
            pl.BlockSpec((1, M), lambda ib: (0, 0)),
            pl.BlockSpec((B_T, OUT), lambda ib: (ib, 0)),
            pl.BlockSpec((M, OUT), lambda ib: (0, 0)),
            pl.BlockSpec((1, 1), lambda ib: (0, 0)),
        ],
        out_specs=[
            pl.BlockSpec((B_T, M), lambda ib: (ib, 0)),
            pl.BlockSpec((B_T, OUT), lambda ib: (ib, 0)),
        ],
        out_shape=[
            jax.ShapeDtypeStruct((B, M), jnp.float32),
            jax.ShapeDtypeStruct((B, OUT), jnp.float32),
        ],
    )(g1, g2, s1, s2, acc, w_min, biases)


def kernel(x, init_struct, beta, gamma, w, biases, chosen_index):
    mean2, var2 = _stats(x)
    mean, var = mean2[0], var2[0]
    inv = jax.lax.rsqrt(var + 0.001)
    scale = gamma * inv
    off = beta - mean * scale
    shiftq = jnp.concatenate(
        [jnp.zeros((1, IN), jnp.float32)]
        + [((c * var + mean) * gamma - beta)[None, :] for c in COEFS], axis=0)
    n1 = chosen_index[:, 1]
    q1 = chosen_index[:, 2]
    n2 = chosen_index[:, 3]
    q2 = chosen_index[:, 4]
    s1 = shiftq[q1, n1][None, :]
    s2 = shiftq[q2, n2][None, :]
    w_q = jnp.transpose(w[:IN * Q].reshape(IN, Q, OUT), (1, 0, 2))
    mx, acc, g1, g2 = _main(x, scale[None, :], off[None, :], shiftq,
                            n1[None, :], n2[None, :], w_q)
    min_x, output = _combine(g1, g2, s1, s2, acc, w[IN * Q:], biases.reshape(1, 1))
    max_x = jnp.transpose(mx, (1, 2, 0))
    return (output, w, max_x, min_x)


# speedup vs baseline: 3.2795x; 1.2084x over previous
import jax
import jax.numpy as jnp
from jax.experimental import pallas as pl
from jax.experimental.pallas import tpu as pltpu

B, IN, Q, M, OUT = 4096, 2048, 6, 512, 16
COEFS = (-3.0, -0.834, -0.248, 0.248, 0.834)

B_T = 256
IN_T = 512


def _stats_body(x_ref, mean_ref, var_ref):
    xb = x_ref[...]
    s = jnp.sum(xb, axis=0)
    ss = jnp.sum(xb * xb, axis=0)
    mean = s * (1.0 / B)
    mean_ref[0, :] = mean
    var_ref[0, :] = ss * (1.0 / B) - mean * mean


def _stats(x):
    return pl.pallas_call(
        _stats_body,
        grid=(IN // 512,),
        in_specs=[pl.BlockSpec((B, 512), lambda i: (0, i))],
        out_specs=[
            pl.BlockSpec((1, 512), lambda i: (0, i)),
            pl.BlockSpec((1, 512), lambda i: (0, i)),
        ],
        out_shape=[
            jax.ShapeDtypeStruct((1, IN), jnp.float32),
            jax.ShapeDtypeStruct((1, IN), jnp.float32),
        ],
    )(x)


def _main_body(x_ref, scale_ref, off_ref, shq_ref, n1_ref, n2_ref,
               wq_ref, mx_ref, acc_ref, g1_ref, g2_ref):
    ii = pl.program_id(1)
    normed = x_ref[...] * scale_ref[0, :] + off_ref[0, :]
    nb = normed.astype(jnp.bfloat16)
    pacc = jnp.zeros((B_T, OUT), jnp.float32)
    for q in range(Q):
        rq = jnp.maximum(normed - shq_ref[q, :], 0.0)
        mx_ref[q] = rq
        pacc += jnp.dot(rq.astype(jnp.bfloat16), wq_ref[q].astype(jnp.bfloat16),
                        preferred_element_type=jnp.float32)
    rows = jax.lax.broadcasted_iota(jnp.int32, (IN_T, M), 0) + ii * IN_T
    p1 = (rows == n1_ref[...]).astype(jnp.bfloat16)
    p2 = (rows == n2_ref[...]).astype(jnp.bfloat16)
    g1p = jnp.dot(nb, p1, preferred_element_type=jnp.float32)
    g2p = jnp.dot(nb, p2, preferred_element_type=jnp.float32)

    @pl.when(ii == 0)
    def _():
        acc_ref[...] = jnp.zeros_like(acc_ref)
        g1_ref[...] = jnp.zeros_like(g1_ref)
        g2_ref[...] = jnp.zeros_like(g2_ref)

    acc_ref[...] += pacc
    g1_ref[...] += g1p
    g2_ref[...] += g2p


def _main(x, scale, off, shiftq, n1, n2, w_q):
    return pl.pallas_call(
        _main_body,
        grid=(B // B_T, IN // IN_T),
        in_specs=[
            pl.BlockSpec((B_T, IN_T), lambda ib, ii: (ib, ii)),
            pl.BlockSpec((1, IN_T), lambda ib, ii: (0, ii)),
            pl.BlockSpec((1, IN_T), lambda ib, ii: (0, ii)),
            pl.BlockSpec((Q, IN_T), lambda ib, ii: (0, ii)),
            pl.BlockSpec((1, M), lambda ib, ii: (0, 0)),
            pl.BlockSpec((1, M), lambda ib, ii: (0, 0)),
            pl.BlockSpec((Q, IN_T, OUT), lambda ib, ii: (0, ii, 0)),
        ],
        out_specs=[
            pl.BlockSpec((Q, B_T, IN_T), lambda ib, ii: (0, ib, ii)),
            pl.BlockSpec((B_T, OUT), lambda ib, ii: (ib, 0)),
            pl.BlockSpec((B_T, M), lambda ib, ii: (ib, 0)),
            pl.BlockSpec((B_T, M), lambda ib, ii: (ib, 0)),
        ],
        out_shape=[
            jax.ShapeDtypeStruct((Q, B, IN), jnp.float32),
            jax.ShapeDtypeStruct((B, OUT), jnp.float32),
            jax.ShapeDtypeStruct((B, M), jnp.float32),
            jax.ShapeDtypeStruct((B, M), jnp.float32),
        ],
        compiler_params=pltpu.CompilerParams(
            dimension_semantics=("parallel", "arbitrary"),
        ),
    )(x, scale, off, shiftq, n1, n2, w_q)


def _combine_body(g1_ref, g2_ref, s1_ref, s2_ref, acc_ref, wmin_ref, b_ref,
                  minx_ref, out_ref):
    d1 = jnp.maximum(g1_ref[...] - s1_ref[0, :], 0.0)
    d2 = jnp.maximum(g2_ref[...] - s2_ref[0, :], 0.0)
    mn = jnp.minimum(d1, d2)
    minx_ref[...] = mn
    out_ref[...] = (acc_ref[...]
                    + jnp.dot(mn, wmin_ref[...], preferred_element_type=jnp.float32)
                    + b_ref[0, 0])


def _combine(g1, g2, s1, s2, acc, w_min, biases):
    return pl.pallas_call(
        _combine_body,
        grid=(B // B_T,),
        in_specs=[
            pl.BlockSpec((B_T, M), lambda ib: (ib, 0)),
            pl.BlockSpec((B_T, M), lambda ib: (ib, 0)),
            pl.BlockSpec((1, M), lambda ib: (0, 0)),
            pl.BlockSpec((1, M), lambda ib: (0, 0)),
            pl.BlockSpec((B_T, OUT), lambda ib: (ib, 0)),
            pl.BlockSpec((M, OUT), lambda ib: (0, 0)),
            pl.BlockSpec((1, 1), lambda ib: (0, 0)),
        ],
        out_specs=[
            pl.BlockSpec((B_T, M), lambda ib: (ib, 0)),
            pl.BlockSpec((B_T, OUT), lambda ib: (ib, 0)),
        ],
        out_shape=[
            jax.ShapeDtypeStruct((B, M), jnp.float32),
            jax.ShapeDtypeStruct((B, OUT), jnp.float32),
        ],
    )(g1, g2, s1, s2, acc, w_min, biases)


def kernel(x, init_struct, beta, gamma, w, biases, chosen_index):
    mean2, var2 = _stats(x)
    mean, var = mean2[0], var2[0]
    inv = jax.lax.rsqrt(var + 0.001)
    scale = gamma * inv
    off = beta - mean * scale
    shiftq = jnp.concatenate(
        [jnp.zeros((1, IN), jnp.float32)]
        + [((c * var + mean) * gamma - beta)[None, :] for c in COEFS], axis=0)
    n1 = chosen_index[:, 1]
    q1 = chosen_index[:, 2]
    n2 = chosen_index[:, 3]
    q2 = chosen_index[:, 4]
    s1 = shiftq[q1, n1][None, :]
    s2 = shiftq[q2, n2][None, :]
    w_q = jnp.transpose(w[:IN * Q].reshape(IN, Q, OUT), (1, 0, 2))
    mx, acc, g1, g2 = _main(x, scale[None, :], off[None, :], shiftq,
                            n1[None, :], n2[None, :], w_q)
    min_x, output = _combine(g1, g2, s1, s2, acc, w[IN * Q:], biases.reshape(1, 1))
    max_x = jnp.transpose(mx, (1, 2, 0))
    return (output, w, max_x, min_x)


# speedup vs baseline: 3.5126x; 1.0711x over previous
import jax
import jax.numpy as jnp
from jax.experimental import pallas as pl
from jax.experimental.pallas import tpu as pltpu

B, IN, Q, M, OUT = 4096, 2048, 6, 512, 16
COEFS = (-3.0, -0.834, -0.248, 0.248, 0.834)

B_T = 256
IN_T = 1024


def _stats_body(x_ref, mean_ref, var_ref):
    xb = x_ref[...]
    s = jnp.sum(xb, axis=0)
    ss = jnp.sum(xb * xb, axis=0)
    mean = s * (1.0 / B)
    mean_ref[0, :] = mean
    var_ref[0, :] = ss * (1.0 / B) - mean * mean


def _stats(x):
    return pl.pallas_call(
        _stats_body,
        grid=(IN // 512,),
        in_specs=[pl.BlockSpec((B, 512), lambda i: (0, i))],
        out_specs=[
            pl.BlockSpec((1, 512), lambda i: (0, i)),
            pl.BlockSpec((1, 512), lambda i: (0, i)),
        ],
        out_shape=[
            jax.ShapeDtypeStruct((1, IN), jnp.float32),
            jax.ShapeDtypeStruct((1, IN), jnp.float32),
        ],
    )(x)


def _main_body(x_ref, scale_ref, off_ref, shq_ref, n1_ref, n2_ref,
               wq_ref, mx_ref, acc_ref, g1_ref, g2_ref):
    ii = pl.program_id(1)
    normed = x_ref[...] * scale_ref[0, :] + off_ref[0, :]
    nb = normed.astype(jnp.bfloat16)
    pacc = jnp.zeros((B_T, OUT), jnp.float32)
    for q in range(Q):
        rq = jnp.maximum(normed - shq_ref[q, :], 0.0)
        mx_ref[q] = rq
        pacc += jnp.dot(rq.astype(jnp.bfloat16), wq_ref[q].astype(jnp.bfloat16),
                        preferred_element_type=jnp.float32)
    rows = jax.lax.broadcasted_iota(jnp.int32, (IN_T, M), 0) + ii * IN_T
    p1 = (rows == n1_ref[...]).astype(jnp.bfloat16)
    p2 = (rows == n2_ref[...]).astype(jnp.bfloat16)
    g1p = jnp.dot(nb, p1, preferred_element_type=jnp.float32)
    g2p = jnp.dot(nb, p2, preferred_element_type=jnp.float32)

    @pl.when(ii == 0)
    def _():
        acc_ref[...] = jnp.zeros_like(acc_ref)
        g1_ref[...] = jnp.zeros_like(g1_ref)
        g2_ref[...] = jnp.zeros_like(g2_ref)

    acc_ref[...] += pacc
    g1_ref[...] += g1p
    g2_ref[...] += g2p


def _main(x, scale, off, shiftq, n1, n2, w_q):
    return pl.pallas_call(
        _main_body,
        grid=(B // B_T, IN // IN_T),
        in_specs=[
            pl.BlockSpec((B_T, IN_T), lambda ib, ii: (ib, ii)),
            pl.BlockSpec((1, IN_T), lambda ib, ii: (0, ii)),
            pl.BlockSpec((1, IN_T), lambda ib, ii: (0, ii)),
            pl.BlockSpec((Q, IN_T), lambda ib, ii: (0, ii)),
            pl.BlockSpec((1, M), lambda ib, ii: (0, 0)),
            pl.BlockSpec((1, M), lambda ib, ii: (0, 0)),
            pl.BlockSpec((Q, IN_T, OUT), lambda ib, ii: (0, ii, 0)),
        ],
        out_specs=[
            pl.BlockSpec((Q, B_T, IN_T), lambda ib, ii: (0, ib, ii)),
            pl.BlockSpec((B_T, OUT), lambda ib, ii: (ib, 0)),
            pl.BlockSpec((B_T, M), lambda ib, ii: (ib, 0)),
            pl.BlockSpec((B_T, M), lambda ib, ii: (ib, 0)),
        ],
        out_shape=[
            jax.ShapeDtypeStruct((Q, B, IN), jnp.float32),
            jax.ShapeDtypeStruct((B, OUT), jnp.float32),
            jax.ShapeDtypeStruct((B, M), jnp.float32),
            jax.ShapeDtypeStruct((B, M), jnp.float32),
        ],
        compiler_params=pltpu.CompilerParams(
            dimension_semantics=("parallel", "arbitrary"),
        ),
    )(x, scale, off, shiftq, n1, n2, w_q)


def _combine_body(g1_ref, g2_ref, s1_ref, s2_ref, acc_ref, wmin_ref, b_ref,
                  minx_ref, out_ref):
    d1 = jnp.maximum(g1_ref[...] - s1_ref[0, :], 0.0)
    d2 = jnp.maximum(g2_ref[...] - s2_ref[0, :], 0.0)
    mn = jnp.minimum(d1, d2)
    minx_ref[...] = mn
    out_ref[...] = (acc_ref[...]
                    + jnp.dot(mn, wmin_ref[...], preferred_element_type=jnp.float32)
                    + b_ref[0, 0])


def _combine(g1, g2, s1, s2, acc, w_min, biases):
    return pl.pallas_call(
        _combine_body,
        grid=(B // B_T,),
        in_specs=[
            pl.BlockSpec((B_T, M), lambda ib: (ib, 0)),
            pl.BlockSpec((B_T, M), lambda ib: (ib, 0)),
            pl.BlockSpec((1, M), lambda ib: (0, 0)),
            pl.BlockSpec((1, M), lambda ib: (0, 0)),
            pl.BlockSpec((B_T, OUT), lambda ib: (ib, 0)),
            pl.BlockSpec((M, OUT), lambda ib: (0, 0)),
            pl.BlockSpec((1, 1), lambda ib: (0, 0)),
        ],
        out_specs=[
            pl.BlockSpec((B_T, M), lambda ib: (ib, 0)),
            pl.BlockSpec((B_T, OUT), lambda ib: (ib, 0)),
        ],
        out_shape=[
            jax.ShapeDtypeStruct((B, M), jnp.float32),
            jax.ShapeDtypeStruct((B, OUT), jnp.float32),
        ],
    )(g1, g2, s1, s2, acc, w_min, biases)


def kernel(x, init_struct, beta, gamma, w, biases, chosen_index):
    mean2, var2 = _stats(x)
    mean, var = mean2[0], var2[0]
    inv = jax.lax.rsqrt(var + 0.001)
    scale = gamma * inv
    off = beta - mean * scale
    shiftq = jnp.concatenate(
        [jnp.zeros((1, IN), jnp.float32)]
        + [((c * var + mean) * gamma - beta)[None, :] for c in COEFS], axis=0)
    n1 = chosen_index[:, 1]
    q1 = chosen_index[:, 2]
    n2 = chosen_index[:, 3]
    q2 = chosen_index[:, 4]
    s1 = shiftq[q1, n1][None, :]
    s2 = shiftq[q2, n2][None, :]
    w_q = jnp.transpose(w[:IN * Q].reshape(IN, Q, OUT), (1, 0, 2))
    mx, acc, g1, g2 = _main(x, scale[None, :], off[None, :], shiftq,
                            n1[None, :], n2[None, :], w_q)
    min_x, output = _combine(g1, g2, s1, s2, acc, w[IN * Q:], biases.reshape(1, 1))
    max_x = jnp.transpose(mx, (1, 2, 0))
    return (output, w, max_x, min_x)


# speedup vs baseline: 4.2263x; 1.2032x over previous
import jax
import jax.numpy as jnp
from jax.experimental import pallas as pl
from jax.experimental.pallas import tpu as pltpu

B, IN, Q, M, OUT = 4096, 2048, 6, 512, 16
COEFS = (-3.0, -0.834, -0.248, 0.248, 0.834)

B_T = 256
IN_T = 2048


def _stats_body(x_ref, mean_ref, var_ref):
    xb = x_ref[...]
    s = jnp.sum(xb, axis=0)
    ss = jnp.sum(xb * xb, axis=0)
    mean = s * (1.0 / B)
    mean_ref[0, :] = mean
    var_ref[0, :] = ss * (1.0 / B) - mean * mean


def _stats(x):
    return pl.pallas_call(
        _stats_body,
        grid=(IN // 512,),
        in_specs=[pl.BlockSpec((B, 512), lambda i: (0, i))],
        out_specs=[
            pl.BlockSpec((1, 512), lambda i: (0, i)),
            pl.BlockSpec((1, 512), lambda i: (0, i)),
        ],
        out_shape=[
            jax.ShapeDtypeStruct((1, IN), jnp.float32),
            jax.ShapeDtypeStruct((1, IN), jnp.float32),
        ],
    )(x)


def _main_body(x_ref, scale_ref, off_ref, shq_ref, n1_ref, n2_ref,
               wq_ref, mx_ref, acc_ref, g1_ref, g2_ref):
    ii = pl.program_id(1)
    normed = x_ref[...] * scale_ref[0, :] + off_ref[0, :]
    nb = normed.astype(jnp.bfloat16)
    pacc = jnp.zeros((B_T, OUT), jnp.float32)
    for q in range(Q):
        rq = jnp.maximum(normed - shq_ref[q, :], 0.0)
        mx_ref[q] = rq
        pacc += jnp.dot(rq.astype(jnp.bfloat16), wq_ref[q].astype(jnp.bfloat16),
                        preferred_element_type=jnp.float32)
    rows = jax.lax.broadcasted_iota(jnp.int32, (IN_T, M), 0) + ii * IN_T
    p1 = (rows == n1_ref[...]).astype(jnp.bfloat16)
    p2 = (rows == n2_ref[...]).astype(jnp.bfloat16)
    g1p = jnp.dot(nb, p1, preferred_element_type=jnp.float32)
    g2p = jnp.dot(nb, p2, preferred_element_type=jnp.float32)

    @pl.when(ii == 0)
    def _():
        acc_ref[...] = jnp.zeros_like(acc_ref)
        g1_ref[...] = jnp.zeros_like(g1_ref)
        g2_ref[...] = jnp.zeros_like(g2_ref)

    acc_ref[...] += pacc
    g1_ref[...] += g1p
    g2_ref[...] += g2p


def _main(x, scale, off, shiftq, n1, n2, w_q):
    return pl.pallas_call(
        _main_body,
        grid=(B // B_T, IN // IN_T),
        in_specs=[
            pl.BlockSpec((B_T, IN_T), lambda ib, ii: (ib, ii)),
            pl.BlockSpec((1, IN_T), lambda ib, ii: (0, ii)),
            pl.BlockSpec((1, IN_T), lambda ib, ii: (0, ii)),
            pl.BlockSpec((Q, IN_T), lambda ib, ii: (0, ii)),
            pl.BlockSpec((1, M), lambda ib, ii: (0, 0)),
            pl.BlockSpec((1, M), lambda ib, ii: (0, 0)),
            pl.BlockSpec((Q, IN_T, OUT), lambda ib, ii: (0, ii, 0)),
        ],
        out_specs=[
            pl.BlockSpec((Q, B_T, IN_T), lambda ib, ii: (0, ib, ii)),
            pl.BlockSpec((B_T, OUT), lambda ib, ii: (ib, 0)),
            pl.BlockSpec((B_T, M), lambda ib, ii: (ib, 0)),
            pl.BlockSpec((B_T, M), lambda ib, ii: (ib, 0)),
        ],
        out_shape=[
            jax.ShapeDtypeStruct((Q, B, IN), jnp.float32),
            jax.ShapeDtypeStruct((B, OUT), jnp.float32),
            jax.ShapeDtypeStruct((B, M), jnp.float32),
            jax.ShapeDtypeStruct((B, M), jnp.float32),
        ],
        compiler_params=pltpu.CompilerParams(
            dimension_semantics=("parallel", "arbitrary"),
        ),
    )(x, scale, off, shiftq, n1, n2, w_q)


def _combine_body(g1_ref, g2_ref, s1_ref, s2_ref, acc_ref, wmin_ref, b_ref,
                  minx_ref, out_ref):
    d1 = jnp.maximum(g1_ref[...] - s1_ref[0, :], 0.0)
    d2 = jnp.maximum(g2_ref[...] - s2_ref[0, :], 0.0)
    mn = jnp.minimum(d1, d2)
    minx_ref[...] = mn
    out_ref[...] = (acc_ref[...]
                    + jnp.dot(mn, wmin_ref[...], preferred_element_type=jnp.float32)
                    + b_ref[0, 0])


def _combine(g1, g2, s1, s2, acc, w_min, biases):
    return pl.pallas_call(
        _combine_body,
        grid=(B // B_T,),
        in_specs=[
            pl.BlockSpec((B_T, M), lambda ib: (ib, 0)),
            pl.BlockSpec((B_T, M), lambda ib: (ib, 0)),
            pl.BlockSpec((1, M), lambda ib: (0, 0)),
            pl.BlockSpec((1, M), lambda ib: (0, 0)),
            pl.BlockSpec((B_T, OUT), lambda ib: (ib, 0)),
            pl.BlockSpec((M, OUT), lambda ib: (0, 0)),
            pl.BlockSpec((1, 1), lambda ib: (0, 0)),
        ],
        out_specs=[
            pl.BlockSpec((B_T, M), lambda ib: (ib, 0)),
            pl.BlockSpec((B_T, OUT), lambda ib: (ib, 0)),
        ],
        out_shape=[
            jax.ShapeDtypeStruct((B, M), jnp.float32),
            jax.ShapeDtypeStruct((B, OUT), jnp.float32),
        ],
    )(g1, g2, s1, s2, acc, w_min, biases)


def kernel(x, init_struct, beta, gamma, w, biases, chosen_index):
    mean2, var2 = _stats(x)
    mean, var = mean2[0], var2[0]
    inv = jax.lax.rsqrt(var + 0.001)
    scale = gamma * inv
    off = beta - mean * scale
    shiftq = jnp.concatenate(
        [jnp.zeros((1, IN), jnp.float32)]
        + [((c * var + mean) * gamma - beta)[None, :] for c in COEFS], axis=0)
    n1 = chosen_index[:, 1]
    q1 = chosen_index[:, 2]
    n2 = chosen_index[:, 3]
    q2 = chosen_index[:, 4]
    s1 = shiftq[q1, n1][None, :]
    s2 = shiftq[q2, n2][None, :]
    w_q = jnp.transpose(w[:IN * Q].reshape(IN, Q, OUT), (1, 0, 2))
    mx, acc, g1, g2 = _main(x, scale[None, :], off[None, :], shiftq,
                            n1[None, :], n2[None, :], w_q)
    min_x, output = _combine(g1, g2, s1, s2, acc, w[IN * Q:], biases.reshape(1, 1))
    max_x = jnp.transpose(mx, (1, 2, 0))
    return (output, w, max_x, min_x)
